# baseline (speedup 1.0000x reference)
import functools

import jax
import jax.numpy as jnp
from jax import lax
from jax.experimental import pallas as pl
from jax.experimental.pallas import tpu as pltpu
from jax.experimental.pallas import tpu_sc as plsc

N = 10000
D = 128
G = 64
NP = 10240
NC = 2
NS = 16
NW = NC * NS
EK = 112
NA = 10112
DRT = NP // NS
RPT = NA // NS
NB = 10
BN = NP // NB


def _deg_body(dst_hbm, w_hbm, deg_out, dst_v, w_v, t640_v, dacc, lsem):
    cid = lax.axis_index("c")
    sid = lax.axis_index("s")
    wid = sid * NC + cid
    ept = dst_hbm.shape[0] // NW
    nch = ept // EK

    def zbody(i, c):
        t640_v[pl.ds(i * 16, 16)] = jnp.zeros((16,), jnp.float32)
        return c
    lax.fori_loop(0, DRT // 16, zbody, 0)
    pltpu.sync_copy(t640_v, dacc.at[pl.ds(sid * DRT, DRT)])
    plsc.subcore_barrier()

    def load(b, g):
        base = wid * ept + g * EK
        pltpu.async_copy(dst_hbm.at[pl.ds(base, EK)], dst_v[b], lsem[b])
        pltpu.async_copy(w_hbm.at[pl.ds(base, EK)], w_v[b], lsem[b])

    def wait_load(b, g):
        base = wid * ept + g * EK
        pltpu.make_async_copy(
            dst_hbm.at[pl.ds(base, EK)], dst_v[b], lsem[b]).wait()
        pltpu.make_async_copy(
            w_hbm.at[pl.ds(base, EK)], w_v[b], lsem[b]).wait()

    def dproc(b, g, prefetch_other, g_other):
        @pl.when(prefetch_other)
        def _():
            load(1 - b, g_other)
        wait_load(b, g)
        pltpu.sync_copy(w_v[b], dacc.at[dst_v[b]], add=True)

    load(0, 0)

    def pair(i, c):
        g0 = 2 * i
        dproc(0, g0, True, g0 + 1)
        dproc(1, g0 + 1, g0 + 2 < nch, g0 + 2)
        return c
    lax.fori_loop(0, nch // 2, pair, 0)
    dproc(0, nch - 1, False, 0)
    plsc.subcore_barrier()

    pltpu.sync_copy(dacc.at[pl.ds(sid * DRT, DRT)], t640_v)
    pltpu.sync_copy(t640_v, deg_out.at[cid, pl.ds(sid * DRT, DRT)])


def _spmm_body(nch0, h_hbm, src_hbm, dst_hbm, w_hbm, s_out,
               src_v, dst_v, w_v, rows_v, gsem, ssem, isem, acc):
    cid = lax.axis_index("c")
    sid = lax.axis_index("s")
    npair = (src_hbm.shape[0] // EK) // NS
    nch = jnp.where(cid == 0, nch0, npair - nch0)
    coff = jnp.where(cid == 0, 0, nch0)
    cbase = sid * npair + coff

    with jax.named_scope("spmm_init"):
        def zrow(e, c):
            for k in range(D // 16):
                rows_v[0][e, pl.ds(k * 16, 16)] = jnp.zeros((16,),
                                                            jnp.float32)
            return c
        lax.fori_loop(0, EK, zrow, 0)
        for j in range(5):
            pltpu.sync_copy(rows_v[0],
                            acc.at[pl.ds(sid * RPT + j * EK, EK)])
        pltpu.sync_copy(rows_v[0].at[pl.ds(0, RPT - 5 * EK)],
                        acc.at[pl.ds(sid * RPT + 5 * EK, RPT - 5 * EK)])
        plsc.subcore_barrier()

    def issue_idx(i6, g):
        base = (cbase + g) * EK
        i3 = i6 % 3
        pltpu.async_copy(src_hbm.at[pl.ds(base, EK)], src_v[i3], isem[i6])
        pltpu.async_copy(dst_hbm.at[pl.ds(base, EK)], dst_v[i6], isem[i6])
        pltpu.async_copy(w_hbm.at[pl.ds(base, EK)], w_v[i3], isem[i6])

    def wait_idx(i6, g):
        base = (cbase + g) * EK
        i3 = i6 % 3
        pltpu.make_async_copy(
            src_hbm.at[pl.ds(base, EK)], src_v[i3], isem[i6]).wait()
        pltpu.make_async_copy(
            dst_hbm.at[pl.ds(base, EK)], dst_v[i6], isem[i6]).wait()
        pltpu.make_async_copy(
            w_hbm.at[pl.ds(base, EK)], w_v[i3], isem[i6]).wait()

    def issue_gather(b3, i6):
        pltpu.async_copy(h_hbm.at[src_v[i6 % 3]], rows_v[b3], gsem[b3])

    def wait_gather(b3, i6):
        pltpu.make_async_copy(h_hbm.at[src_v[i6 % 3]], rows_v[b3],
                              gsem[b3]).wait()

    def issue_scatter(b3, i6):
        pltpu.async_copy(rows_v[b3], acc.at[dst_v[i6]], ssem[b3], add=True)

    def wait_scatter(b3, i6):
        pltpu.make_async_copy(rows_v[b3], acc.at[dst_v[i6]],
                              ssem[b3]).wait()

    def scale(b3, i6):
        def srow(g16, cc):
            w16 = w_v[i6 % 3][pl.ds(g16 * 16, 16)]
            for j in range(16):
                e = g16 * 16 + j
                ws = w16[j]
                for k in range(D // 16):
                    sl = pl.ds(k * 16, 16)
                    rows_v[b3][e, sl] = rows_v[b3][e, sl] * ws
            return cc
        lax.fori_loop(0, EK // 16, srow, 0)

    def chunk_step(g, q):
        b, i = q % 3, q
        bp1, ip1 = (q + 1) % 3, (q + 1) % 6
        bm2, im2 = (q - 2) % 3, (q - 2) % 6

        @pl.when(g >= 2)
        def _():
            wait_scatter(bm2, im2)
        @pl.when(g + 1 < nch)
        def _():
            wait_idx(ip1, g + 1)
            issue_gather(bp1, ip1)
        wait_gather(b, i)
        scale(b, i)
        issue_scatter(b, i)
        @pl.when(g + 2 < nch)
        def _():
            issue_idx((q + 2) % 6, g + 2)

    with jax.named_scope("spmm_edges"):
        issue_idx(0, 0)
        issue_idx(1, 1)
        wait_idx(0, 0)
        issue_gather(0, 0)

        def six(n, c):
            for q in range(6):
                chunk_step(6 * n + q, q)
            return c
        lax.fori_loop(0, (nch - 1) // 6, six, 0)
        chunk_step(nch - 1, 0)
        wait_scatter(2, 5)
        wait_scatter(0, 0)
        plsc.subcore_barrier()

    with jax.named_scope("spmm_writeout"):
        for j in range(5):
            r0 = sid * RPT + j * EK
            pltpu.sync_copy(acc.at[pl.ds(r0, EK)], rows_v[0])
            pltpu.sync_copy(rows_v[0], s_out.at[cid, pl.ds(r0, EK)])
        r0 = sid * RPT + 5 * EK
        rem = RPT - 5 * EK
        pltpu.sync_copy(acc.at[pl.ds(r0, rem)], rows_v[0].at[pl.ds(0, rem)])
        pltpu.sync_copy(rows_v[0].at[pl.ds(0, rem)],
                        s_out.at[cid, pl.ds(r0, rem)])


_SC_MESH = plsc.VectorSubcoreMesh(core_axis_name="c", subcore_axis_name="s")

_deg_call = pl.kernel(
    _deg_body,
    out_type=jax.ShapeDtypeStruct((NC, NP), jnp.float32),
    mesh=_SC_MESH,
    scratch_types=[
        (pltpu.VMEM((EK,), jnp.int32), pltpu.VMEM((EK,), jnp.int32)),
        (pltpu.VMEM((EK,), jnp.float32), pltpu.VMEM((EK,), jnp.float32)),
        pltpu.VMEM((DRT,), jnp.float32),
        pltpu.VMEM_SHARED((NP,), jnp.float32),
        (pltpu.SemaphoreType.DMA, pltpu.SemaphoreType.DMA),
    ],
)

def _make_spmm(nch0):
  return pl.kernel(
    functools.partial(_spmm_body, nch0),
    out_type=jax.ShapeDtypeStruct((NC, NA, D), jnp.float32),
    mesh=_SC_MESH,
    scratch_types=[
        tuple(pltpu.VMEM((EK,), jnp.int32) for _ in range(3)),
        tuple(pltpu.VMEM((EK,), jnp.int32) for _ in range(6)),
        tuple(pltpu.VMEM((EK,), jnp.float32) for _ in range(3)),
        tuple(pltpu.VMEM((EK, D), jnp.float32) for _ in range(3)),
        tuple(pltpu.SemaphoreType.DMA for _ in range(3)),
        tuple(pltpu.SemaphoreType.DMA for _ in range(3)),
        tuple(pltpu.SemaphoreType.DMA for _ in range(6)),
        pltpu.VMEM_SHARED((NA, D), jnp.float32),
    ],
  )


_spmm_call_l1 = _make_spmm(127)
_spmm_call_l2 = _make_spmm(121)


def _tc1_body(x_ref, w1_ref, deg_ref, h1p_ref, dis_ref):
    degs = deg_ref[0] + deg_ref[1] + 1.0
    s = jnp.where(degs > 0, lax.rsqrt(jnp.where(degs > 0, degs, 1.0)), 0.0)
    h = jnp.dot(x_ref[...], w1_ref[...], preferred_element_type=jnp.float32)
    h1p_ref[...] = h * s
    dis_ref[...] = s


def _tc2_body(s1_ref, h1p_ref, dis_ref, b1_ref, w2_ref, h2p_ref):
    s = dis_ref[...]
    pre = (s1_ref[0] + s1_ref[1] + h1p_ref[...]) * s + b1_ref[...]
    o = jnp.maximum(pre, 0.0)
    h2 = jnp.dot(o, w2_ref[...], preferred_element_type=jnp.float32)
    h2p_ref[...] = h2 * s


def _tc3_body(s2_ref, h2p_ref, dis_ref, b2_ref, batch_ref, out_ref):
    i = pl.program_id(0)
    s = dis_ref[...]
    pre = (s2_ref[0] + s2_ref[1] + h2p_ref[...]) * s + b2_ref[...]
    o = jnp.maximum(pre, 0.0)
    ridx = i * BN + lax.broadcasted_iota(jnp.int32, (BN, 1), 0)
    o = jnp.where(ridx < NA, o, 0.0)
    seg = lax.broadcasted_iota(jnp.int32, (BN, G), 1)
    oh = (batch_ref[...] == seg).astype(jnp.float32)
    contrib = lax.dot_general(oh, o, (((0,), (0,)), ((), ())),
                              preferred_element_type=jnp.float32)

    @pl.when(i == 0)
    def _init():
        out_ref[...] = contrib

    @pl.when(i > 0)
    def _acc():
        out_ref[...] += contrib


_tc1_call = pl.pallas_call(
    _tc1_body,
    grid=(NB,),
    in_specs=[
        pl.BlockSpec((BN, D), lambda i: (i, 0)),
        pl.BlockSpec((D, D), lambda i: (0, 0)),
        pl.BlockSpec((NC, BN, 1), lambda i: (0, i, 0)),
    ],
    out_specs=[
        pl.BlockSpec((BN, D), lambda i: (i, 0)),
        pl.BlockSpec((BN, 1), lambda i: (i, 0)),
    ],
    out_shape=[
        jax.ShapeDtypeStruct((NP, D), jnp.float32),
        jax.ShapeDtypeStruct((NP, 1), jnp.float32),
    ],
)

_tc2_call = pl.pallas_call(
    _tc2_body,
    grid=(NB,),
    in_specs=[
        pl.BlockSpec((NC, BN, D), lambda i: (0, i, 0)),
        pl.BlockSpec((BN, D), lambda i: (i, 0)),
        pl.BlockSpec((BN, 1), lambda i: (i, 0)),
        pl.BlockSpec((1, D), lambda i: (0, 0)),
        pl.BlockSpec((D, D), lambda i: (0, 0)),
    ],
    out_specs=pl.BlockSpec((BN, D), lambda i: (i, 0)),
    out_shape=jax.ShapeDtypeStruct((NP, D), jnp.float32),
)

_tc3_call = pl.pallas_call(
    _tc3_body,
    grid=(NB,),
    in_specs=[
        pl.BlockSpec((NC, BN, D), lambda i: (0, i, 0)),
        pl.BlockSpec((BN, D), lambda i: (i, 0)),
        pl.BlockSpec((BN, 1), lambda i: (i, 0)),
        pl.BlockSpec((1, D), lambda i: (0, 0)),
        pl.BlockSpec((BN, 1), lambda i: (i, 0)),
    ],
    out_specs=pl.BlockSpec((G, D), lambda i: (0, 0)),
    out_shape=jax.ShapeDtypeStruct((G, D), jnp.float32),
)


@jax.jit
def kernel(x, edge_index, edge_weight, batch, W1, b1, W2, b2):
    e = edge_weight.shape[0]
    pp = -(-e // (NS * EK))
    pp += (2 - pp) % 6
    epad = pp * NS * EK
    pe = epad - e

    src = edge_index[0].astype(jnp.int32)
    dst = edge_index[1].astype(jnp.int32)
    src_p = jnp.concatenate([src, jnp.zeros((pe,), jnp.int32)])
    dst_p = jnp.concatenate([dst, jnp.zeros((pe,), jnp.int32)])
    w_p = jnp.concatenate([edge_weight, jnp.zeros((pe,), edge_weight.dtype)])

    x_p = jnp.concatenate([x, jnp.zeros((NP - N, D), x.dtype)])
    batch_p = jnp.concatenate(
        [batch.astype(jnp.int32), jnp.full((NP - N,), G, jnp.int32)]
    ).reshape(NP, 1)

    deg2 = _deg_call(dst_p, w_p).reshape(NC, NP, 1)
    h1p, dis = _tc1_call(x_p, W1, deg2)
    s1 = _spmm_call_l1(h1p, src_p, dst_p, w_p)
    h2p = _tc2_call(s1, h1p, dis, b1.reshape(1, D), W2)
    s2 = _spmm_call_l2(h2p, src_p, dst_p, w_p)
    return _tc3_call(s2, h2p, dis, b2.reshape(1, D), batch_p)


# speedup vs baseline: 1.0927x; 1.0927x over previous
import functools

import jax
import jax.numpy as jnp
from jax import lax
from jax.experimental import pallas as pl
from jax.experimental.pallas import tpu as pltpu
from jax.experimental.pallas import tpu_sc as plsc

N = 10000
D = 128
G = 64
NP = 10240
NC = 2
NS = 16
NW = NC * NS
EK = 112
NA = 10112
DRT = NP // NS
RPT = NA // NS
NB = 10
BN = NP // NB


def _deg_body(dst_hbm, w_hbm, deg_out, dst_v, w_v, t640_v, dacc, lsem):
    cid = lax.axis_index("c")
    sid = lax.axis_index("s")
    wid = sid * NC + cid
    ept = dst_hbm.shape[0] // NW
    nch = ept // EK

    def zbody(i, c):
        t640_v[pl.ds(i * 16, 16)] = jnp.zeros((16,), jnp.float32)
        return c
    lax.fori_loop(0, DRT // 16, zbody, 0)
    pltpu.sync_copy(t640_v, dacc.at[pl.ds(sid * DRT, DRT)])
    plsc.subcore_barrier()

    def load(b, g):
        base = wid * ept + g * EK
        pltpu.async_copy(dst_hbm.at[pl.ds(base, EK)], dst_v[b], lsem[b])
        pltpu.async_copy(w_hbm.at[pl.ds(base, EK)], w_v[b], lsem[b])

    def wait_load(b, g):
        base = wid * ept + g * EK
        pltpu.make_async_copy(
            dst_hbm.at[pl.ds(base, EK)], dst_v[b], lsem[b]).wait()
        pltpu.make_async_copy(
            w_hbm.at[pl.ds(base, EK)], w_v[b], lsem[b]).wait()

    def dproc(b, g, prefetch_other, g_other):
        @pl.when(prefetch_other)
        def _():
            load(1 - b, g_other)
        wait_load(b, g)
        pltpu.sync_copy(w_v[b], dacc.at[dst_v[b]], add=True)

    load(0, 0)

    def pair(i, c):
        g0 = 2 * i
        dproc(0, g0, True, g0 + 1)
        dproc(1, g0 + 1, g0 + 2 < nch, g0 + 2)
        return c
    lax.fori_loop(0, nch // 2, pair, 0)
    dproc(0, nch - 1, False, 0)
    plsc.subcore_barrier()

    pltpu.sync_copy(dacc.at[pl.ds(sid * DRT, DRT)], t640_v)
    pltpu.sync_copy(t640_v, deg_out.at[cid, pl.ds(sid * DRT, DRT)])


def _spmm_body(nch0, h_hbm, src_hbm, dst_hbm, w_hbm, s_out,
               src_v, dst_v, w_v, rows_v, gsem, ssem, isem, acc):
    cid = lax.axis_index("c")
    sid = lax.axis_index("s")
    npair = (src_hbm.shape[0] // EK) // NS
    nch = jnp.where(cid == 0, nch0, npair - nch0)
    coff = jnp.where(cid == 0, 0, nch0)
    cbase = sid * npair + coff

    with jax.named_scope("spmm_init"):
        def zrow(e, c):
            for k in range(D // 16):
                rows_v[0][e, pl.ds(k * 16, 16)] = jnp.zeros((16,),
                                                            jnp.float32)
            return c
        lax.fori_loop(0, EK, zrow, 0)
        for j in range(5):
            pltpu.sync_copy(rows_v[0],
                            acc.at[pl.ds(sid * RPT + j * EK, EK)])
        pltpu.sync_copy(rows_v[0].at[pl.ds(0, RPT - 5 * EK)],
                        acc.at[pl.ds(sid * RPT + 5 * EK, RPT - 5 * EK)])
        plsc.subcore_barrier()

    def issue_idx(i6, g):
        base = (cbase + g) * EK
        i3 = i6 % 3
        pltpu.async_copy(src_hbm.at[pl.ds(base, EK)], src_v[i3], isem[i6])
        pltpu.async_copy(dst_hbm.at[pl.ds(base, EK)], dst_v[i6], isem[i6])
        pltpu.async_copy(w_hbm.at[pl.ds(base, EK)], w_v[i3], isem[i6])

    def wait_idx(i6, g):
        base = (cbase + g) * EK
        i3 = i6 % 3
        pltpu.make_async_copy(
            src_hbm.at[pl.ds(base, EK)], src_v[i3], isem[i6]).wait()
        pltpu.make_async_copy(
            dst_hbm.at[pl.ds(base, EK)], dst_v[i6], isem[i6]).wait()
        pltpu.make_async_copy(
            w_hbm.at[pl.ds(base, EK)], w_v[i3], isem[i6]).wait()

    def issue_gather(b3, i6):
        pltpu.async_copy(h_hbm.at[src_v[i6 % 3]], rows_v[b3], gsem[b3])

    def wait_gather(b3, i6):
        pltpu.make_async_copy(h_hbm.at[src_v[i6 % 3]], rows_v[b3],
                              gsem[b3]).wait()

    def issue_scatter(b3, i6):
        pltpu.async_copy(rows_v[b3], acc.at[dst_v[i6]], ssem[b3], add=True)

    def wait_scatter(b3, i6):
        pltpu.make_async_copy(rows_v[b3], acc.at[dst_v[i6]],
                              ssem[b3]).wait()

    def scale(b3, i6):
        def srow(g16, cc):
            w16 = w_v[i6 % 3][pl.ds(g16 * 16, 16)]
            for j in range(16):
                e = g16 * 16 + j
                ws = w16[j]
                for k in range(D // 16):
                    sl = pl.ds(k * 16, 16)
                    rows_v[b3][e, sl] = rows_v[b3][e, sl] * ws
            return cc
        lax.fori_loop(0, EK // 16, srow, 0)

    def chunk_step(g, q):
        b, i = q % 3, q
        bp1, ip1 = (q + 1) % 3, (q + 1) % 6
        bm2, im2 = (q - 2) % 3, (q - 2) % 6

        @pl.when(g >= 2)
        def _():
            wait_scatter(bm2, im2)
        @pl.when(g + 1 < nch)
        def _():
            wait_idx(ip1, g + 1)
            issue_gather(bp1, ip1)
        wait_gather(b, i)
        scale(b, i)
        issue_scatter(b, i)
        @pl.when(g + 2 < nch)
        def _():
            issue_idx((q + 2) % 6, g + 2)

    with jax.named_scope("spmm_edges"):
        issue_idx(0, 0)
        issue_idx(1, 1)
        wait_idx(0, 0)
        issue_gather(0, 0)

        def six(n, c):
            for q in range(6):
                chunk_step(6 * n + q, q)
            return c
        lax.fori_loop(0, (nch - 1) // 6, six, 0)
        chunk_step(nch - 1, 0)
        wait_scatter(2, 5)
        wait_scatter(0, 0)
        plsc.subcore_barrier()

    with jax.named_scope("spmm_writeout"):
        for j in range(5):
            r0 = sid * RPT + j * EK
            pltpu.sync_copy(acc.at[pl.ds(r0, EK)], rows_v[0])
            pltpu.sync_copy(rows_v[0], s_out.at[cid, pl.ds(r0, EK)])
        r0 = sid * RPT + 5 * EK
        rem = RPT - 5 * EK
        pltpu.sync_copy(acc.at[pl.ds(r0, rem)], rows_v[0].at[pl.ds(0, rem)])
        pltpu.sync_copy(rows_v[0].at[pl.ds(0, rem)],
                        s_out.at[cid, pl.ds(r0, rem)])


_SC_MESH = plsc.VectorSubcoreMesh(core_axis_name="c", subcore_axis_name="s")

_deg_call = pl.kernel(
    _deg_body,
    out_type=jax.ShapeDtypeStruct((NC, NP), jnp.float32),
    mesh=_SC_MESH,
    scratch_types=[
        (pltpu.VMEM((EK,), jnp.int32), pltpu.VMEM((EK,), jnp.int32)),
        (pltpu.VMEM((EK,), jnp.float32), pltpu.VMEM((EK,), jnp.float32)),
        pltpu.VMEM((DRT,), jnp.float32),
        pltpu.VMEM_SHARED((NP,), jnp.float32),
        (pltpu.SemaphoreType.DMA, pltpu.SemaphoreType.DMA),
    ],
)

def _make_spmm(nch0):
  return pl.kernel(
    functools.partial(_spmm_body, nch0),
    out_type=jax.ShapeDtypeStruct((NC, NA, D), jnp.float32),
    mesh=_SC_MESH,
    scratch_types=[
        tuple(pltpu.VMEM((EK,), jnp.int32) for _ in range(3)),
        tuple(pltpu.VMEM((EK,), jnp.int32) for _ in range(6)),
        tuple(pltpu.VMEM((EK,), jnp.float32) for _ in range(3)),
        tuple(pltpu.VMEM((EK, D), jnp.float32) for _ in range(3)),
        tuple(pltpu.SemaphoreType.DMA for _ in range(3)),
        tuple(pltpu.SemaphoreType.DMA for _ in range(3)),
        tuple(pltpu.SemaphoreType.DMA for _ in range(6)),
        pltpu.VMEM_SHARED((NA, D), jnp.float32),
    ],
  )


_spmm_call_l1 = _make_spmm(145)
_spmm_call_l2 = _make_spmm(145)


def _tc1_body(x_ref, w1_ref, deg_ref, h1p_ref, dis_ref):
    degs = deg_ref[0] + deg_ref[1] + 1.0
    s = jnp.where(degs > 0, lax.rsqrt(jnp.where(degs > 0, degs, 1.0)), 0.0)
    h = jnp.dot(x_ref[...], w1_ref[...], preferred_element_type=jnp.float32)
    h1p_ref[...] = h * s
    dis_ref[...] = s


def _tc2_body(s1_ref, h1p_ref, dis_ref, b1_ref, w2_ref, h2p_ref):
    s = dis_ref[...]
    pre = (s1_ref[0] + s1_ref[1] + h1p_ref[...]) * s + b1_ref[...]
    o = jnp.maximum(pre, 0.0)
    h2 = jnp.dot(o, w2_ref[...], preferred_element_type=jnp.float32)
    h2p_ref[...] = h2 * s


def _tc3_body(s2_ref, h2p_ref, dis_ref, b2_ref, batch_ref, out_ref):
    i = pl.program_id(0)
    s = dis_ref[...]
    pre = (s2_ref[0] + s2_ref[1] + h2p_ref[...]) * s + b2_ref[...]
    o = jnp.maximum(pre, 0.0)
    ridx = i * BN + lax.broadcasted_iota(jnp.int32, (BN, 1), 0)
    o = jnp.where(ridx < NA, o, 0.0)
    seg = lax.broadcasted_iota(jnp.int32, (BN, G), 1)
    oh = (batch_ref[...] == seg).astype(jnp.float32)
    contrib = lax.dot_general(oh, o, (((0,), (0,)), ((), ())),
                              preferred_element_type=jnp.float32)

    @pl.when(i == 0)
    def _init():
        out_ref[...] = contrib

    @pl.when(i > 0)
    def _acc():
        out_ref[...] += contrib


_tc1_call = pl.pallas_call(
    _tc1_body,
    grid=(NB,),
    in_specs=[
        pl.BlockSpec((BN, D), lambda i: (i, 0)),
        pl.BlockSpec((D, D), lambda i: (0, 0)),
        pl.BlockSpec((NC, BN, 1), lambda i: (0, i, 0)),
    ],
    out_specs=[
        pl.BlockSpec((BN, D), lambda i: (i, 0)),
        pl.BlockSpec((BN, 1), lambda i: (i, 0)),
    ],
    out_shape=[
        jax.ShapeDtypeStruct((NP, D), jnp.float32),
        jax.ShapeDtypeStruct((NP, 1), jnp.float32),
    ],
)

_tc2_call = pl.pallas_call(
    _tc2_body,
    grid=(NB,),
    in_specs=[
        pl.BlockSpec((NC, BN, D), lambda i: (0, i, 0)),
        pl.BlockSpec((BN, D), lambda i: (i, 0)),
        pl.BlockSpec((BN, 1), lambda i: (i, 0)),
        pl.BlockSpec((1, D), lambda i: (0, 0)),
        pl.BlockSpec((D, D), lambda i: (0, 0)),
    ],
    out_specs=pl.BlockSpec((BN, D), lambda i: (i, 0)),
    out_shape=jax.ShapeDtypeStruct((NP, D), jnp.float32),
)

_tc3_call = pl.pallas_call(
    _tc3_body,
    grid=(NB,),
    in_specs=[
        pl.BlockSpec((NC, BN, D), lambda i: (0, i, 0)),
        pl.BlockSpec((BN, D), lambda i: (i, 0)),
        pl.BlockSpec((BN, 1), lambda i: (i, 0)),
        pl.BlockSpec((1, D), lambda i: (0, 0)),
        pl.BlockSpec((BN, 1), lambda i: (i, 0)),
    ],
    out_specs=pl.BlockSpec((G, D), lambda i: (0, 0)),
    out_shape=jax.ShapeDtypeStruct((G, D), jnp.float32),
)


@jax.jit
def kernel(x, edge_index, edge_weight, batch, W1, b1, W2, b2):
    e = edge_weight.shape[0]
    pp = -(-e // (NS * EK))
    pp += (2 - pp) % 6
    epad = pp * NS * EK
    pe = epad - e

    src = edge_index[0].astype(jnp.int32)
    dst = edge_index[1].astype(jnp.int32)
    src_p = jnp.concatenate([src, jnp.zeros((pe,), jnp.int32)])
    dst_p = jnp.concatenate([dst, jnp.zeros((pe,), jnp.int32)])
    w_p = jnp.concatenate([edge_weight, jnp.zeros((pe,), edge_weight.dtype)])

    x_p = jnp.concatenate([x, jnp.zeros((NP - N, D), x.dtype)])
    batch_p = jnp.concatenate(
        [batch.astype(jnp.int32), jnp.full((NP - N,), G, jnp.int32)]
    ).reshape(NP, 1)

    deg2 = _deg_call(dst_p, w_p).reshape(NC, NP, 1)
    h1p, dis = _tc1_call(x_p, W1, deg2)
    s1 = _spmm_call_l1(h1p, src_p, dst_p, w_p)
    h2p = _tc2_call(s1, h1p, dis, b1.reshape(1, D), W2)
    s2 = _spmm_call_l2(h2p, src_p, dst_p, w_p)
    return _tc3_call(s2, h2p, dis, b2.reshape(1, D), batch_p)


# speedup vs baseline: 1.3840x; 1.2666x over previous
import functools

import jax
import jax.numpy as jnp
from jax import lax
from jax.experimental import pallas as pl
from jax.experimental.pallas import tpu as pltpu
from jax.experimental.pallas import tpu_sc as plsc

N = 10000
D = 128
G = 64
NP = 10240
NC = 2
NS = 16
NW = NC * NS
EK = 128
NA = NP
DRT = NP // NS
RPT = NA // NS
RCH = RPT // EK
NB = 10
BN = NP // NB


def _deg_body(dst_hbm, w_hbm, deg_out, dst_v, w_v, t640_v, dacc, lsem):
    cid = lax.axis_index("c")
    sid = lax.axis_index("s")
    wid = sid * NC + cid
    ept = dst_hbm.shape[0] // NW
    nch = ept // EK

    def zbody(i, c):
        t640_v[pl.ds(i * 16, 16)] = jnp.zeros((16,), jnp.float32)
        return c
    lax.fori_loop(0, DRT // 16, zbody, 0)
    pltpu.sync_copy(t640_v, dacc.at[pl.ds(sid * DRT, DRT)])
    plsc.subcore_barrier()

    def load(b, g):
        base = wid * ept + g * EK
        pltpu.async_copy(dst_hbm.at[pl.ds(base, EK)], dst_v[b], lsem[b])
        pltpu.async_copy(w_hbm.at[pl.ds(base, EK)], w_v[b], lsem[b])

    def wait_load(b, g):
        base = wid * ept + g * EK
        pltpu.make_async_copy(
            dst_hbm.at[pl.ds(base, EK)], dst_v[b], lsem[b]).wait()
        pltpu.make_async_copy(
            w_hbm.at[pl.ds(base, EK)], w_v[b], lsem[b]).wait()

    def dproc(b, g, prefetch_other, g_other):
        @pl.when(prefetch_other)
        def _():
            load(1 - b, g_other)
        wait_load(b, g)
        pltpu.sync_copy(w_v[b], dacc.at[dst_v[b]], add=True)

    load(0, 0)

    def pair(i, c):
        g0 = 2 * i
        dproc(0, g0, True, g0 + 1)
        dproc(1, g0 + 1, g0 + 2 < nch, g0 + 2)
        return c
    lax.fori_loop(0, nch // 2, pair, 0)
    dproc(0, nch - 1, False, 0)
    plsc.subcore_barrier()

    pltpu.sync_copy(dacc.at[pl.ds(sid * DRT, DRT)], t640_v)
    pltpu.sync_copy(t640_v, deg_out.at[cid, pl.ds(sid * DRT, DRT)])


def _spmm_body(nch0, h_hbm, src_hbm, dst_hbm, w_hbm, s_out,
               src_v, dst_v, w_v, rows_v, gsem, ssem, isem, acc):
    cid = lax.axis_index("c")
    sid = lax.axis_index("s")
    npair = (src_hbm.shape[0] // EK) // NS
    nch = jnp.where(cid == 0, nch0, npair - nch0)
    coff = jnp.where(cid == 0, 0, nch0)
    cbase = sid * npair + coff

    with jax.named_scope("spmm_init"):
        def zrow(e, c):
            for k in range(D // 16):
                rows_v[0][e, pl.ds(k * 16, 16)] = jnp.zeros((16,),
                                                            jnp.float32)
            return c
        lax.fori_loop(0, EK, zrow, 0)
        for j in range(RCH):
            pltpu.sync_copy(rows_v[0], acc.at[pl.ds(sid * RPT + j * EK, EK)])
        plsc.subcore_barrier()

    def issue_idx(i4, g):
        base = (cbase + g) * EK
        pltpu.async_copy(src_hbm.at[pl.ds(base, EK)], src_v[i4], isem[i4])
        pltpu.async_copy(dst_hbm.at[pl.ds(base, EK)], dst_v[i4], isem[i4])
        pltpu.async_copy(w_hbm.at[pl.ds(base, EK)], w_v[i4], isem[i4])

    def wait_idx(i4, g):
        base = (cbase + g) * EK
        pltpu.make_async_copy(
            src_hbm.at[pl.ds(base, EK)], src_v[i4], isem[i4]).wait()
        pltpu.make_async_copy(
            dst_hbm.at[pl.ds(base, EK)], dst_v[i4], isem[i4]).wait()
        pltpu.make_async_copy(
            w_hbm.at[pl.ds(base, EK)], w_v[i4], isem[i4]).wait()

    def issue_gather(b, i4):
        pltpu.async_copy(h_hbm.at[src_v[i4]], rows_v[b], gsem[b])

    def wait_gather(b, i4):
        pltpu.make_async_copy(h_hbm.at[src_v[i4]], rows_v[b],
                              gsem[b]).wait()

    def issue_scatter(b, i4):
        pltpu.async_copy(rows_v[b], acc.at[dst_v[i4]], ssem[b], add=True)

    def wait_scatter(b, i4):
        pltpu.make_async_copy(rows_v[b], acc.at[dst_v[i4]],
                              ssem[b]).wait()

    def scale(b, i4):
        def srow(g16, cc):
            w16 = w_v[i4][pl.ds(g16 * 16, 16)]
            for j in range(16):
                e = g16 * 16 + j
                ws = w16[j]
                for k in range(D // 16):
                    sl = pl.ds(k * 16, 16)
                    rows_v[b][e, sl] = rows_v[b][e, sl] * ws
            return cc
        lax.fori_loop(0, EK // 16, srow, 0)

    def chunk_step(g, q):
        b, i = q % 2, q
        bp1, ip1 = (q + 1) % 2, (q + 1) % 4
        im2 = (q - 2) % 4

        @pl.when(g + 1 < nch)
        def _():
            wait_idx(ip1, g + 1)
            issue_gather(bp1, ip1)
        @pl.when(g >= 2)
        def _():
            wait_scatter(b, im2)
        wait_gather(b, i)
        scale(b, i)
        issue_scatter(b, i)
        @pl.when(g + 2 < nch)
        def _():
            issue_idx((q + 2) % 4, g + 2)

    with jax.named_scope("spmm_edges"):
        issue_idx(0, 0)
        issue_idx(1, 1)
        wait_idx(0, 0)
        issue_gather(0, 0)

        def four(n, c):
            for q in range(4):
                chunk_step(4 * n + q, q)
            return c
        lax.fori_loop(0, (nch - 1) // 4, four, 0)
        chunk_step(nch - 1, 0)
        wait_scatter(1, 3)
        wait_scatter(0, 0)
        plsc.subcore_barrier()

    with jax.named_scope("spmm_writeout"):
        for j in range(RCH):
            r0 = sid * RPT + j * EK
            pltpu.sync_copy(acc.at[pl.ds(r0, EK)], rows_v[0])
            pltpu.sync_copy(rows_v[0], s_out.at[cid, pl.ds(r0, EK)])


_SC_MESH = plsc.VectorSubcoreMesh(core_axis_name="c", subcore_axis_name="s")

_deg_call = pl.kernel(
    _deg_body,
    out_type=jax.ShapeDtypeStruct((NC, NP), jnp.float32),
    mesh=_SC_MESH,
    scratch_types=[
        (pltpu.VMEM((EK,), jnp.int32), pltpu.VMEM((EK,), jnp.int32)),
        (pltpu.VMEM((EK,), jnp.float32), pltpu.VMEM((EK,), jnp.float32)),
        pltpu.VMEM((DRT,), jnp.float32),
        pltpu.VMEM_SHARED((NP,), jnp.float32),
        (pltpu.SemaphoreType.DMA, pltpu.SemaphoreType.DMA),
    ],
)

def _make_spmm(nch0):
  return pl.kernel(
    functools.partial(_spmm_body, nch0),
    out_type=jax.ShapeDtypeStruct((NC, NA, D), jnp.float32),
    mesh=_SC_MESH,
    scratch_types=[
        tuple(pltpu.VMEM((EK,), jnp.int32) for _ in range(4)),
        tuple(pltpu.VMEM((EK,), jnp.int32) for _ in range(4)),
        tuple(pltpu.VMEM((EK,), jnp.float32) for _ in range(4)),
        tuple(pltpu.VMEM((EK, D), jnp.float32) for _ in range(2)),
        tuple(pltpu.SemaphoreType.DMA for _ in range(2)),
        tuple(pltpu.SemaphoreType.DMA for _ in range(2)),
        tuple(pltpu.SemaphoreType.DMA for _ in range(4)),
        pltpu.VMEM_SHARED((NA, D), jnp.float32),
    ],
  )


_spmm_call_l1 = _make_spmm(121)
_spmm_call_l2 = _make_spmm(121)


def _tc1_body(x_ref, w1_ref, deg_ref, h1p_ref, dis_ref):
    degs = deg_ref[0] + deg_ref[1] + 1.0
    s = jnp.where(degs > 0, lax.rsqrt(jnp.where(degs > 0, degs, 1.0)), 0.0)
    h = jnp.dot(x_ref[...], w1_ref[...], preferred_element_type=jnp.float32)
    h1p_ref[...] = h * s
    dis_ref[...] = s


def _tc2_body(s1_ref, h1p_ref, dis_ref, b1_ref, w2_ref, h2p_ref):
    s = dis_ref[...]
    pre = (s1_ref[0] + s1_ref[1] + h1p_ref[...]) * s + b1_ref[...]
    o = jnp.maximum(pre, 0.0)
    h2 = jnp.dot(o, w2_ref[...], preferred_element_type=jnp.float32)
    h2p_ref[...] = h2 * s


def _tc3_body(s2_ref, h2p_ref, dis_ref, b2_ref, batch_ref, out_ref):
    i = pl.program_id(0)
    s = dis_ref[...]
    pre = (s2_ref[0] + s2_ref[1] + h2p_ref[...]) * s + b2_ref[...]
    o = jnp.maximum(pre, 0.0)
    ridx = i * BN + lax.broadcasted_iota(jnp.int32, (BN, 1), 0)
    o = jnp.where(ridx < NA, o, 0.0)
    seg = lax.broadcasted_iota(jnp.int32, (BN, G), 1)
    oh = (batch_ref[...] == seg).astype(jnp.float32)
    contrib = lax.dot_general(oh, o, (((0,), (0,)), ((), ())),
                              preferred_element_type=jnp.float32)

    @pl.when(i == 0)
    def _init():
        out_ref[...] = contrib

    @pl.when(i > 0)
    def _acc():
        out_ref[...] += contrib


_tc1_call = pl.pallas_call(
    _tc1_body,
    grid=(NB,),
    in_specs=[
        pl.BlockSpec((BN, D), lambda i: (i, 0)),
        pl.BlockSpec((D, D), lambda i: (0, 0)),
        pl.BlockSpec((NC, BN, 1), lambda i: (0, i, 0)),
    ],
    out_specs=[
        pl.BlockSpec((BN, D), lambda i: (i, 0)),
        pl.BlockSpec((BN, 1), lambda i: (i, 0)),
    ],
    out_shape=[
        jax.ShapeDtypeStruct((NP, D), jnp.float32),
        jax.ShapeDtypeStruct((NP, 1), jnp.float32),
    ],
)

_tc2_call = pl.pallas_call(
    _tc2_body,
    grid=(NB,),
    in_specs=[
        pl.BlockSpec((NC, BN, D), lambda i: (0, i, 0)),
        pl.BlockSpec((BN, D), lambda i: (i, 0)),
        pl.BlockSpec((BN, 1), lambda i: (i, 0)),
        pl.BlockSpec((1, D), lambda i: (0, 0)),
        pl.BlockSpec((D, D), lambda i: (0, 0)),
    ],
    out_specs=pl.BlockSpec((BN, D), lambda i: (i, 0)),
    out_shape=jax.ShapeDtypeStruct((NP, D), jnp.float32),
)

_tc3_call = pl.pallas_call(
    _tc3_body,
    grid=(NB,),
    in_specs=[
        pl.BlockSpec((NC, BN, D), lambda i: (0, i, 0)),
        pl.BlockSpec((BN, D), lambda i: (i, 0)),
        pl.BlockSpec((BN, 1), lambda i: (i, 0)),
        pl.BlockSpec((1, D), lambda i: (0, 0)),
        pl.BlockSpec((BN, 1), lambda i: (i, 0)),
    ],
    out_specs=pl.BlockSpec((G, D), lambda i: (0, 0)),
    out_shape=jax.ShapeDtypeStruct((G, D), jnp.float32),
)


@jax.jit
def kernel(x, edge_index, edge_weight, batch, W1, b1, W2, b2):
    e = edge_weight.shape[0]
    pp = -(-e // (NS * EK))
    pp += (2 - pp) % 4
    epad = pp * NS * EK
    pe = epad - e

    src = edge_index[0].astype(jnp.int32)
    dst = edge_index[1].astype(jnp.int32)
    src_p = jnp.concatenate([src, jnp.zeros((pe,), jnp.int32)])
    dst_p = jnp.concatenate([dst, jnp.zeros((pe,), jnp.int32)])
    w_p = jnp.concatenate([edge_weight, jnp.zeros((pe,), edge_weight.dtype)])

    x_p = jnp.concatenate([x, jnp.zeros((NP - N, D), x.dtype)])
    batch_p = jnp.concatenate(
        [batch.astype(jnp.int32), jnp.full((NP - N,), G, jnp.int32)]
    ).reshape(NP, 1)

    deg2 = _deg_call(dst_p, w_p).reshape(NC, NP, 1)
    h1p, dis = _tc1_call(x_p, W1, deg2)
    s1 = _spmm_call_l1(h1p, src_p, dst_p, w_p)
    h2p = _tc2_call(s1, h1p, dis, b1.reshape(1, D), W2)
    s2 = _spmm_call_l2(h2p, src_p, dst_p, w_p)
    return _tc3_call(s2, h2p, dis, b2.reshape(1, D), batch_p)


# speedup vs baseline: 1.4142x; 1.0218x over previous
import functools

import jax
import jax.numpy as jnp
from jax import lax
from jax.experimental import pallas as pl
from jax.experimental.pallas import tpu as pltpu
from jax.experimental.pallas import tpu_sc as plsc

N = 10000
D = 128
G = 64
NP = 10240
NC = 2
NS = 16
NW = NC * NS
EK = 128
NA = NP
DRT = NP // NS
RPT = NA // NS
RCH = RPT // EK
NB = 10
BN = NP // NB


def _deg_body(dst_hbm, w_hbm, deg_out, dst_v, w_v, t640_v, dacc, lsem):
    cid = lax.axis_index("c")
    sid = lax.axis_index("s")
    wid = sid * NC + cid
    ept = dst_hbm.shape[0] // NW
    nch = ept // EK

    def zbody(i, c):
        t640_v[pl.ds(i * 16, 16)] = jnp.zeros((16,), jnp.float32)
        return c
    lax.fori_loop(0, DRT // 16, zbody, 0)
    pltpu.sync_copy(t640_v, dacc.at[pl.ds(sid * DRT, DRT)])
    plsc.subcore_barrier()

    def load(b, g):
        base = wid * ept + g * EK
        pltpu.async_copy(dst_hbm.at[pl.ds(base, EK)], dst_v[b], lsem[b])
        pltpu.async_copy(w_hbm.at[pl.ds(base, EK)], w_v[b], lsem[b])

    def wait_load(b, g):
        base = wid * ept + g * EK
        pltpu.make_async_copy(
            dst_hbm.at[pl.ds(base, EK)], dst_v[b], lsem[b]).wait()
        pltpu.make_async_copy(
            w_hbm.at[pl.ds(base, EK)], w_v[b], lsem[b]).wait()

    def dproc(b, g, prefetch_other, g_other):
        @pl.when(prefetch_other)
        def _():
            load(1 - b, g_other)
        wait_load(b, g)
        pltpu.sync_copy(w_v[b], dacc.at[dst_v[b]], add=True)

    load(0, 0)

    def pair(i, c):
        g0 = 2 * i
        dproc(0, g0, True, g0 + 1)
        dproc(1, g0 + 1, g0 + 2 < nch, g0 + 2)
        return c
    lax.fori_loop(0, nch // 2, pair, 0)
    dproc(0, nch - 1, False, 0)
    plsc.subcore_barrier()

    pltpu.sync_copy(dacc.at[pl.ds(sid * DRT, DRT)], t640_v)
    pltpu.sync_copy(t640_v, deg_out.at[cid, pl.ds(sid * DRT, DRT)])


def _spmm_body(nch0, h_hbm, src_hbm, dst_hbm, w_hbm, s_out,
               src_v, dst_v, w_v, rows_v, gsem, ssem, isem, acc):
    cid = lax.axis_index("c")
    sid = lax.axis_index("s")
    npair = (src_hbm.shape[0] // EK) // NS
    nch = jnp.where(cid == 0, nch0, npair - nch0)
    coff = jnp.where(cid == 0, 0, nch0)
    cbase = sid * npair + coff

    with jax.named_scope("spmm_init"):
        def zrow(e, c):
            for k in range(D // 16):
                rows_v[0][e, pl.ds(k * 16, 16)] = jnp.zeros((16,),
                                                            jnp.float32)
            return c
        lax.fori_loop(0, EK, zrow, 0)
        for j in range(RCH):
            pltpu.sync_copy(rows_v[0], acc.at[pl.ds(sid * RPT + j * EK, EK)])
        plsc.subcore_barrier()

    def issue_idx(i4, g):
        base = (cbase + g) * EK
        pltpu.async_copy(src_hbm.at[pl.ds(base, EK)], src_v[i4], isem[i4])
        pltpu.async_copy(dst_hbm.at[pl.ds(base, EK)], dst_v[i4], isem[i4])
        pltpu.async_copy(w_hbm.at[pl.ds(base, EK)], w_v[i4], isem[i4])

    def wait_idx(i4, g):
        base = (cbase + g) * EK
        pltpu.make_async_copy(
            src_hbm.at[pl.ds(base, EK)], src_v[i4], isem[i4]).wait()
        pltpu.make_async_copy(
            dst_hbm.at[pl.ds(base, EK)], dst_v[i4], isem[i4]).wait()
        pltpu.make_async_copy(
            w_hbm.at[pl.ds(base, EK)], w_v[i4], isem[i4]).wait()

    def issue_gather(b, i4):
        pltpu.async_copy(h_hbm.at[src_v[i4]], rows_v[b], gsem[b])

    def wait_gather(b, i4):
        pltpu.make_async_copy(h_hbm.at[src_v[i4]], rows_v[b],
                              gsem[b]).wait()

    def issue_scatter(b, i4):
        pltpu.async_copy(rows_v[b], acc.at[dst_v[i4]], ssem[b], add=True)

    def wait_scatter(b, i4):
        pltpu.make_async_copy(rows_v[b], acc.at[dst_v[i4]],
                              ssem[b]).wait()

    def scale(b, i4):
        def srow(g16, cc):
            w16 = w_v[i4][pl.ds(g16 * 16, 16)]
            for j in range(16):
                e = g16 * 16 + j
                ws = w16[j]
                for k in range(D // 16):
                    sl = pl.ds(k * 16, 16)
                    rows_v[b][e, sl] = rows_v[b][e, sl] * ws
            return cc
        lax.fori_loop(0, EK // 16, srow, 0)

    def chunk_step(g, q):
        b, i = q % 2, q
        bp1, ip1 = (q + 1) % 2, (q + 1) % 4
        im2 = (q - 2) % 4

        @pl.when(g + 1 < nch)
        def _():
            wait_idx(ip1, g + 1)
            issue_gather(bp1, ip1)
        @pl.when(g >= 2)
        def _():
            wait_scatter(b, im2)
        wait_gather(b, i)
        scale(b, i)
        issue_scatter(b, i)
        @pl.when(g + 2 < nch)
        def _():
            issue_idx((q + 2) % 4, g + 2)

    with jax.named_scope("spmm_edges"):
        issue_idx(0, 0)
        issue_idx(1, 1)
        wait_idx(0, 0)
        issue_gather(0, 0)

        def four(n, c):
            for q in range(4):
                chunk_step(4 * n + q, q)
            return c
        lax.fori_loop(0, (nch - 1) // 4, four, 0)
        chunk_step(nch - 1, 0)
        wait_scatter(1, 3)
        wait_scatter(0, 0)
        plsc.subcore_barrier()

    with jax.named_scope("spmm_writeout"):
        for j in range(RCH):
            r0 = sid * RPT + j * EK
            pltpu.sync_copy(acc.at[pl.ds(r0, EK)], rows_v[0])
            pltpu.sync_copy(rows_v[0], s_out.at[cid, pl.ds(r0, EK)])


_SC_MESH = plsc.VectorSubcoreMesh(core_axis_name="c", subcore_axis_name="s")

_deg_call = pl.kernel(
    _deg_body,
    out_type=jax.ShapeDtypeStruct((NC, NP), jnp.float32),
    mesh=_SC_MESH,
    scratch_types=[
        (pltpu.VMEM((EK,), jnp.int32), pltpu.VMEM((EK,), jnp.int32)),
        (pltpu.VMEM((EK,), jnp.float32), pltpu.VMEM((EK,), jnp.float32)),
        pltpu.VMEM((DRT,), jnp.float32),
        pltpu.VMEM_SHARED((NP,), jnp.float32),
        (pltpu.SemaphoreType.DMA, pltpu.SemaphoreType.DMA),
    ],
)

def _make_spmm(nch0):
  return pl.kernel(
    functools.partial(_spmm_body, nch0),
    out_type=jax.ShapeDtypeStruct((NC, NA, D), jnp.float32),
    mesh=_SC_MESH,
    scratch_types=[
        tuple(pltpu.VMEM((EK,), jnp.int32) for _ in range(4)),
        tuple(pltpu.VMEM((EK,), jnp.int32) for _ in range(4)),
        tuple(pltpu.VMEM((EK,), jnp.float32) for _ in range(4)),
        tuple(pltpu.VMEM((EK, D), jnp.float32) for _ in range(2)),
        tuple(pltpu.SemaphoreType.DMA for _ in range(2)),
        tuple(pltpu.SemaphoreType.DMA for _ in range(2)),
        tuple(pltpu.SemaphoreType.DMA for _ in range(4)),
        pltpu.VMEM_SHARED((NA, D), jnp.float32),
    ],
  )


_spmm_call_l1 = _make_spmm(129)
_spmm_call_l2 = _make_spmm(121)


def _tc1_body(x_ref, w1_ref, deg_ref, h1p_ref, dis_ref):
    degs = deg_ref[0] + deg_ref[1] + 1.0
    s = jnp.where(degs > 0, lax.rsqrt(jnp.where(degs > 0, degs, 1.0)), 0.0)
    h = jnp.dot(x_ref[...], w1_ref[...], preferred_element_type=jnp.float32)
    h1p_ref[...] = h * s
    dis_ref[...] = s


def _tc2_body(s1_ref, h1p_ref, dis_ref, b1_ref, w2_ref, h2p_ref):
    s = dis_ref[...]
    pre = (s1_ref[0] + s1_ref[1] + h1p_ref[...]) * s + b1_ref[...]
    o = jnp.maximum(pre, 0.0)
    h2 = jnp.dot(o, w2_ref[...], preferred_element_type=jnp.float32)
    h2p_ref[...] = h2 * s


def _tc3_body(s2_ref, h2p_ref, dis_ref, b2_ref, batch_ref, out_ref):
    i = pl.program_id(0)
    s = dis_ref[...]
    pre = (s2_ref[0] + s2_ref[1] + h2p_ref[...]) * s + b2_ref[...]
    o = jnp.maximum(pre, 0.0)
    ridx = i * BN + lax.broadcasted_iota(jnp.int32, (BN, 1), 0)
    o = jnp.where(ridx < NA, o, 0.0)
    seg = lax.broadcasted_iota(jnp.int32, (BN, G), 1)
    oh = (batch_ref[...] == seg).astype(jnp.float32)
    contrib = lax.dot_general(oh, o, (((0,), (0,)), ((), ())),
                              preferred_element_type=jnp.float32)

    @pl.when(i == 0)
    def _init():
        out_ref[...] = contrib

    @pl.when(i > 0)
    def _acc():
        out_ref[...] += contrib


_tc1_call = pl.pallas_call(
    _tc1_body,
    grid=(NB,),
    in_specs=[
        pl.BlockSpec((BN, D), lambda i: (i, 0)),
        pl.BlockSpec((D, D), lambda i: (0, 0)),
        pl.BlockSpec((NC, BN, 1), lambda i: (0, i, 0)),
    ],
    out_specs=[
        pl.BlockSpec((BN, D), lambda i: (i, 0)),
        pl.BlockSpec((BN, 1), lambda i: (i, 0)),
    ],
    out_shape=[
        jax.ShapeDtypeStruct((NP, D), jnp.float32),
        jax.ShapeDtypeStruct((NP, 1), jnp.float32),
    ],
)

_tc2_call = pl.pallas_call(
    _tc2_body,
    grid=(NB,),
    in_specs=[
        pl.BlockSpec((NC, BN, D), lambda i: (0, i, 0)),
        pl.BlockSpec((BN, D), lambda i: (i, 0)),
        pl.BlockSpec((BN, 1), lambda i: (i, 0)),
        pl.BlockSpec((1, D), lambda i: (0, 0)),
        pl.BlockSpec((D, D), lambda i: (0, 0)),
    ],
    out_specs=pl.BlockSpec((BN, D), lambda i: (i, 0)),
    out_shape=jax.ShapeDtypeStruct((NP, D), jnp.float32),
)

_tc3_call = pl.pallas_call(
    _tc3_body,
    grid=(NB,),
    in_specs=[
        pl.BlockSpec((NC, BN, D), lambda i: (0, i, 0)),
        pl.BlockSpec((BN, D), lambda i: (i, 0)),
        pl.BlockSpec((BN, 1), lambda i: (i, 0)),
        pl.BlockSpec((1, D), lambda i: (0, 0)),
        pl.BlockSpec((BN, 1), lambda i: (i, 0)),
    ],
    out_specs=pl.BlockSpec((G, D), lambda i: (0, 0)),
    out_shape=jax.ShapeDtypeStruct((G, D), jnp.float32),
)


@jax.jit
def kernel(x, edge_index, edge_weight, batch, W1, b1, W2, b2):
    e = edge_weight.shape[0]
    pp = -(-e // (NS * EK))
    pp += (2 - pp) % 4
    epad = pp * NS * EK
    pe = epad - e

    src = edge_index[0].astype(jnp.int32)
    dst = edge_index[1].astype(jnp.int32)
    src_p = jnp.concatenate([src, jnp.zeros((pe,), jnp.int32)])
    dst_p = jnp.concatenate([dst, jnp.zeros((pe,), jnp.int32)])
    w_p = jnp.concatenate([edge_weight, jnp.zeros((pe,), edge_weight.dtype)])

    x_p = jnp.concatenate([x, jnp.zeros((NP - N, D), x.dtype)])
    batch_p = jnp.concatenate(
        [batch.astype(jnp.int32), jnp.full((NP - N,), G, jnp.int32)]
    ).reshape(NP, 1)

    deg2 = _deg_call(dst_p, w_p).reshape(NC, NP, 1)
    h1p, dis = _tc1_call(x_p, W1, deg2)
    s1 = _spmm_call_l1(h1p, src_p, dst_p, w_p)
    h2p = _tc2_call(s1, h1p, dis, b1.reshape(1, D), W2)
    s2 = _spmm_call_l2(h2p, src_p, dst_p, w_p)
    return _tc3_call(s2, h2p, dis, b2.reshape(1, D), batch_p)
